# baseline (speedup 1.0000x reference)
import functools

import jax
import jax.numpy as jnp
from jax import lax
from jax.experimental import pallas as pl
from jax.experimental.pallas import tpu as pltpu
from jax.experimental.pallas import tpu_sc as plsc

BATCH = 16384
N_CONT = 13
N_CAT = 26
VOCAB = 100000
EMB = 64

NUM_CORES = 2
NUM_SUBCORES = 16
NW = NUM_CORES * NUM_SUBCORES
ROWS_PER_W = BATCH // NW
BLOCK = 64
N_BLOCKS = ROWS_PER_W // BLOCK
TOTAL_BLOCKS = N_CAT * N_BLOCKS


def _make_kernel():
    mesh = plsc.VectorSubcoreMesh(core_axis_name="c", subcore_axis_name="s")

    @functools.partial(
        pl.kernel,
        mesh=mesh,
        out_type=jax.ShapeDtypeStruct((BATCH, N_CAT * EMB), jnp.float32),
        compiler_params=pltpu.CompilerParams(use_tc_tiling_on_sc=False),
        scratch_types=[
            pltpu.VMEM((N_CAT, ROWS_PER_W), jnp.int32),
            pltpu.SemaphoreType.DMA,
        ],
    )
    def emb_kernel(xcat_hbm, table_hbm, out_hbm, idx_v, sem):
        wid = lax.axis_index("s") * NUM_CORES + lax.axis_index("c")
        base = wid * ROWS_PER_W
        pltpu.sync_copy(xcat_hbm.at[wid], idx_v)

        def drain_block():
            pltpu.make_async_copy(
                table_hbm.at[0, pl.ds(0, BLOCK), :],
                out_hbm.at[pl.ds(0, BLOCK), pl.ds(0, EMB)],
                sem,
            ).wait()

        def per_block(t, carry):
            c = t // N_BLOCKS
            b = t % N_BLOCKS
            for v in range(BLOCK // 16):
                off = b * BLOCK + v * 16
                vec = idx_v[c, pl.ds(off, 16)]
                for k in range(16):
                    pltpu.async_copy(
                        table_hbm.at[c, vec[k]],
                        out_hbm.at[base + off + k, pl.ds(c * EMB, EMB)],
                        sem,
                    )

            @pl.when(t >= 2)
            def _():
                drain_block()

            return carry

        lax.fori_loop(0, TOTAL_BLOCKS, per_block, 0)
        drain_block()
        drain_block()

    return emb_kernel


_emb_kernel = _make_kernel()


def kernel(x, tables):
    xcat = x[:, N_CONT:].reshape(NW, ROWS_PER_W, N_CAT).transpose(0, 2, 1)
    emb = _emb_kernel(xcat, tables)
    xcont = x[:, :N_CONT].astype(jnp.float32)
    return jnp.concatenate([xcont, emb], axis=1)


# speedup vs baseline: 4.1125x; 4.1125x over previous
import functools

import jax
import jax.numpy as jnp
from jax import lax
from jax.experimental import pallas as pl
from jax.experimental.pallas import tpu as pltpu
from jax.experimental.pallas import tpu_sc as plsc

BATCH = 16384
N_CONT = 13
N_CAT = 26
VOCAB = 100000
EMB = 64

NUM_CORES = 2
NUM_SUBCORES = 16
NW = NUM_CORES * NUM_SUBCORES
ROWS_PER_W = BATCH // NW
BLOCK = 64
HALF = ROWS_PER_W // 2
HALF_BLOCKS = HALF // BLOCK


def _make_kernel():
    mesh = plsc.VectorSubcoreMesh(core_axis_name="c", subcore_axis_name="s")

    @functools.partial(
        pl.kernel,
        mesh=mesh,
        out_type=jax.ShapeDtypeStruct((N_CAT, BATCH, EMB), jnp.float32),
        scratch_types=[
            pltpu.VMEM((N_CAT, ROWS_PER_W), jnp.int32),
            pltpu.VMEM((2, HALF, EMB), jnp.float32),
            pltpu.SemaphoreType.DMA,
            pltpu.SemaphoreType.DMA,
        ],
    )
    def emb_kernel(xcat_hbm, table_hbm, out_hbm, idx_v, rows_v, gsem, wsem):
        wid = lax.axis_index("s") * NUM_CORES + lax.axis_index("c")
        base = wid * ROWS_PER_W
        pltpu.sync_copy(xcat_hbm.at[wid], idx_v)

        def drain_gather_block():
            pltpu.make_async_copy(
                table_hbm.at[0, pl.ds(0, BLOCK), :],
                rows_v.at[0, pl.ds(0, BLOCK), :],
                gsem,
            ).wait()

        def drain_out_write():
            pltpu.make_async_copy(
                rows_v.at[0],
                out_hbm.at[0, pl.ds(0, HALF), :],
                wsem,
            ).wait()

        def per_half(t, carry):
            c = t // 2
            row0 = (t % 2) * HALF
            buf = t % 2

            @pl.when(t >= 2)
            def _():
                drain_out_write()

            def per_block(b, carry2):
                for v in range(BLOCK // 16):
                    off = b * BLOCK + v * 16
                    vec = idx_v[c, pl.ds(row0 + off, 16)]
                    for k in range(16):
                        pltpu.async_copy(
                            table_hbm.at[c, vec[k]],
                            rows_v.at[buf, off + k],
                            gsem,
                        )

                @pl.when(b >= 1)
                def _():
                    drain_gather_block()

                return carry2

            lax.fori_loop(0, HALF_BLOCKS, per_block, 0)
            drain_gather_block()
            pltpu.async_copy(
                rows_v.at[buf],
                out_hbm.at[c, pl.ds(base + row0, HALF), :],
                wsem,
            )
            return carry

        lax.fori_loop(0, 2 * N_CAT, per_half, 0)
        drain_out_write()
        drain_out_write()

    return emb_kernel


_emb_kernel = _make_kernel()


def kernel(x, tables):
    xcat = x[:, N_CONT:].reshape(NW, ROWS_PER_W, N_CAT).transpose(0, 2, 1)
    emb = _emb_kernel(xcat, tables)
    xcont = x[:, :N_CONT].astype(jnp.float32)
    return jnp.concatenate(
        [xcont, emb.transpose(1, 0, 2).reshape(BATCH, N_CAT * EMB)], axis=1)
